# Initial kernel scaffold; baseline (speedup 1.0000x reference)
import jax
import jax.numpy as jnp
from jax.experimental import pallas as pl


def kernel(x, edge_index, root_n_index, W1, b1, W2, b2, P1, pb1, P2, pb2):
    raise NotImplementedError("write your pallas kernel here")


# speedup vs baseline: 9.6787x; 9.6787x over previous
import jax
import jax.numpy as jnp
from jax import lax
from jax.experimental import pallas as pl
from jax.experimental.pallas import tpu as pltpu
from jax.experimental.pallas import tpu_sc as plsc

N_NODES = 10000
D = 128
NC = 2
NS = 16
NW = NC * NS
CHUNK = 128
N_PAD = 10240
ROWS_PER_TILE = N_PAD // NS
ZROWS = 64
GR = N_PAD // D
GPT = GR // NS
N_LINKS = 1024
LPW = N_LINKS // NW

_MESH = dict(core_axis_name="c", subcore_axis_name="s")


def _wid():
    return lax.axis_index("s") * NC + lax.axis_index("c")


def _agg_body(h_hbm, src_hbm, dst_hbm, zrow_hbm, out_hbm,
              isv, idv, rows, zv, sem0, acc):
    k = src_hbm.shape[1]
    c = lax.axis_index("c")
    s = lax.axis_index("s")
    w = _wid()
    base = s * ROWS_PER_TILE

    pltpu.sync_copy(zrow_hbm, zv)

    def zbody(r, carry):
        pltpu.sync_copy(zv, acc.at[pl.ds(base + r * ZROWS, ZROWS), :])
        return carry

    lax.fori_loop(0, ROWS_PER_TILE // ZROWS, zbody, 0)
    plsc.subcore_barrier()

    def body(j, carry):
        pltpu.sync_copy(src_hbm.at[w, j], isv.at[0])
        pltpu.sync_copy(dst_hbm.at[w, j], idv.at[0])
        pltpu.async_copy(h_hbm.at[isv.at[0]], rows.at[0], sem0).wait()
        pltpu.sync_copy(rows.at[0], acc.at[idv.at[0]], add=True)
        return carry

    lax.fori_loop(0, k, body, 0)
    plsc.subcore_barrier()

    def obody(r, carry):
        sl = pl.ds(base + r * ZROWS, ZROWS)
        pltpu.sync_copy(acc.at[sl, :], zv)
        pltpu.sync_copy(zv, out_hbm.at[c, sl, :])
        return carry

    lax.fori_loop(0, ROWS_PER_TILE // ZROWS, obody, 0)


def _link_body(h_hbm, sd_hbm, out_hbm, sdv, rows, prod, sem):
    w = _wid()
    pltpu.sync_copy(sd_hbm.at[w], sdv.at[0])
    pltpu.async_copy(h_hbm.at[sdv.at[0]], rows, sem).wait()
    for p in range(LPW):
        for t in range(D // 16):
            sl = pl.ds(t * 16, 16)
            prod[p, sl] = rows[p, sl] * rows[p + LPW, sl]
    pltpu.sync_copy(prod, out_hbm.at[pl.ds(w * LPW, LPW), :])


_RB = 1024


_EB = 2048


def _hist_body(dstr_ref, dstc_ref, o_ref, acc_ref):
    i = pl.program_id(0)

    @pl.when(i == 0)
    def _():
        acc_ref[...] = jnp.zeros_like(acc_ref)

    rr = dstr_ref[...]
    cc = dstc_ref[...]
    ri = lax.broadcasted_iota(jnp.int32, (1, GR), 1)
    ci = lax.broadcasted_iota(jnp.int32, (1, D), 1)
    R = (rr == ri).astype(jnp.float32)
    C = (cc == ci).astype(jnp.float32)
    acc_ref[...] += lax.dot_general(
        R, C, (((0,), (0,)), ((), ())),
        preferred_element_type=jnp.float32)

    @pl.when(i == pl.num_programs(0) - 1)
    def _():
        node = (lax.broadcasted_iota(jnp.int32, (GR, D), 0) * D
                + lax.broadcasted_iota(jnp.int32, (GR, D), 1))
        dg = acc_ref[...] + (node < N_NODES).astype(jnp.float32)
        o_ref[...] = jnp.where(dg > 0, lax.rsqrt(dg), 0.0)


def _mm1_body(x_ref, w_ref, dinv_ref, o_ref):
    i = pl.program_id(0)
    h = jnp.dot(x_ref[...], w_ref[...], preferred_element_type=jnp.float32)
    rows = i * _RB + lax.broadcasted_iota(jnp.int32, (_RB, 1), 0)
    o_ref[...] = jnp.where(rows < N_NODES, h * dinv_ref[...], 0.0)


def _mid_body(aggp_ref, dinv_ref, b1_ref, w2_ref, o_ref):
    i = pl.program_id(0)
    dinv = dinv_ref[...]
    agg = aggp_ref[0] + aggp_ref[1]
    h1 = jnp.maximum(agg * dinv + b1_ref[...], 0.0)
    h = jnp.dot(h1, w2_ref[...], preferred_element_type=jnp.float32) * dinv
    rows = i * _RB + lax.broadcasted_iota(jnp.int32, (_RB, 1), 0)
    o_ref[...] = jnp.where(rows < N_NODES, h, 0.0)


def _fin_body(aggp_ref, dinv_ref, b2_ref, o_ref):
    o_ref[...] = (aggp_ref[0] + aggp_ref[1]) * dinv_ref[...] + b2_ref[...]


def _head_body(link_ref, p1_ref, pb1_ref, p2_ref, pb2_ref, o_ref):
    t = jnp.dot(link_ref[...], p1_ref[...],
                preferred_element_type=jnp.float32) + pb1_ref[...]
    t = jnp.maximum(t, 0.0)
    o_ref[...] = jnp.dot(t, p2_ref[...],
                         preferred_element_type=jnp.float32) + pb2_ref[0, 0]


def kernel(x, edge_index, root_n_index, W1, b1, W2, b2, P1, pb1, P2, pb2):
    n = x.shape[0]
    e_real = edge_index.shape[1] + n
    k = -(-e_real // (NW * CHUNK))
    k += k % 2
    e_pad = NW * CHUNK * k
    n_extra = N_PAD - n

    loop = jnp.arange(n, dtype=jnp.int32)
    src = jnp.concatenate([edge_index[0].astype(jnp.int32), loop])
    dst = jnp.concatenate([edge_index[1].astype(jnp.int32), loop])
    pad_idx = n + (jnp.arange(e_pad - e_real, dtype=jnp.int32) % n_extra)
    src3 = jnp.concatenate([src, pad_idx]).reshape(NW, k, CHUNK)
    dst3 = jnp.concatenate([dst, pad_idx]).reshape(NW, k, CHUNK)

    x_pad = jnp.pad(x, ((0, n_extra), (0, 0)))
    zgrid = jnp.zeros((GR, D), jnp.float32)
    zrow = jnp.zeros((ZROWS, D), jnp.float32)
    iota80 = jnp.arange(GR, dtype=jnp.int32).reshape(1, GR)
    lpad = n + (jnp.arange(NW * 2 * LPW, dtype=jnp.int32) % n_extra)
    sd = jnp.concatenate(
        [root_n_index[:, 0].astype(jnp.int32).reshape(NW, LPW),
         root_n_index[:, 1].astype(jnp.int32).reshape(NW, LPW),
         lpad.reshape(NW, 2 * LPW)], axis=1)

    mesh = plsc.VectorSubcoreMesh(**_MESH)

    e_d = edge_index.shape[1]
    kd = -(-e_d // _EB)
    dpad = n + (jnp.arange(kd * _EB - e_d, dtype=jnp.int32) % n_extra)
    dd = jnp.concatenate([edge_index[1].astype(jnp.int32), dpad])
    dinv_grid = pl.pallas_call(
        _hist_body,
        grid=(kd,),
        in_specs=[
            pl.BlockSpec((_EB, 1), lambda i: (i, 0)),
            pl.BlockSpec((_EB, 1), lambda i: (i, 0)),
        ],
        out_specs=pl.BlockSpec((GR, D), lambda i: (0, 0)),
        out_shape=jax.ShapeDtypeStruct((GR, D), jnp.float32),
        scratch_shapes=[pltpu.VMEM((GR, D), jnp.float32)],
    )((dd >> 7).reshape(-1, 1), (dd & 127).reshape(-1, 1))
    dinv_col = dinv_grid.reshape(N_PAD, 1)

    agg_call = pl.kernel(
        _agg_body,
        out_type=jax.ShapeDtypeStruct((NC, N_PAD, D), jnp.float32),
        mesh=mesh,
        scratch_types=[
            pltpu.VMEM((2, CHUNK), jnp.int32),
            pltpu.VMEM((2, CHUNK), jnp.int32),
            pltpu.VMEM((2, CHUNK, D), jnp.float32),
            pltpu.VMEM((ZROWS, D), jnp.float32),
            pltpu.SemaphoreType.DMA,
            pltpu.VMEM_SHARED((N_PAD, D), jnp.float32),
        ],
    )

    grid = (N_PAD // _RB,)
    h1p = pl.pallas_call(
        _mm1_body,
        grid=grid,
        in_specs=[
            pl.BlockSpec((_RB, D), lambda i: (i, 0)),
            pl.BlockSpec((D, D), lambda i: (0, 0)),
            pl.BlockSpec((_RB, 1), lambda i: (i, 0)),
        ],
        out_specs=pl.BlockSpec((_RB, D), lambda i: (i, 0)),
        out_shape=jax.ShapeDtypeStruct((N_PAD, D), jnp.float32),
    )(x_pad, W1, dinv_col)

    agg1 = agg_call(h1p, src3, dst3, zrow)

    h2p = pl.pallas_call(
        _mid_body,
        grid=grid,
        in_specs=[
            pl.BlockSpec((NC, _RB, D), lambda i: (0, i, 0)),
            pl.BlockSpec((_RB, 1), lambda i: (i, 0)),
            pl.BlockSpec((1, D), lambda i: (0, 0)),
            pl.BlockSpec((D, D), lambda i: (0, 0)),
        ],
        out_specs=pl.BlockSpec((_RB, D), lambda i: (i, 0)),
        out_shape=jax.ShapeDtypeStruct((N_PAD, D), jnp.float32),
    )(agg1, dinv_col, b1.reshape(1, D), W2)

    agg2 = agg_call(h2p, src3, dst3, zrow)

    h2 = pl.pallas_call(
        _fin_body,
        grid=grid,
        in_specs=[
            pl.BlockSpec((NC, _RB, D), lambda i: (0, i, 0)),
            pl.BlockSpec((_RB, 1), lambda i: (i, 0)),
            pl.BlockSpec((1, D), lambda i: (0, 0)),
        ],
        out_specs=pl.BlockSpec((_RB, D), lambda i: (i, 0)),
        out_shape=jax.ShapeDtypeStruct((N_PAD, D), jnp.float32),
    )(agg2, dinv_col, b2.reshape(1, D))

    link = pl.kernel(
        _link_body,
        out_type=jax.ShapeDtypeStruct((N_LINKS, D), jnp.float32),
        mesh=mesh,
        scratch_types=[
            pltpu.VMEM((1, 4 * LPW), jnp.int32),
            pltpu.VMEM((4 * LPW, D), jnp.float32),
            pltpu.VMEM((LPW, D), jnp.float32),
            pltpu.SemaphoreType.DMA,
        ],
    )(h2, sd)

    P2pad = jnp.pad(P2, ((0, 0), (0, D - P2.shape[1])))
    logits2d = pl.pallas_call(
        _head_body,
        in_specs=[
            pl.BlockSpec((N_LINKS, D), lambda: (0, 0)),
            pl.BlockSpec((D, D), lambda: (0, 0)),
            pl.BlockSpec((1, D), lambda: (0, 0)),
            pl.BlockSpec((D, D), lambda: (0, 0)),
            pl.BlockSpec((1, 1), lambda: (0, 0)),
        ],
        out_specs=pl.BlockSpec((N_LINKS, D), lambda: (0, 0)),
        out_shape=jax.ShapeDtypeStruct((N_LINKS, D), jnp.float32),
    )(link, P1, pb1.reshape(1, D), P2pad, pb2.reshape(1, 1))

    return logits2d[:, 0]


# speedup vs baseline: 12.0807x; 1.2482x over previous
import jax
import jax.numpy as jnp
from jax import lax
from jax.experimental import pallas as pl
from jax.experimental.pallas import tpu as pltpu
from jax.experimental.pallas import tpu_sc as plsc

N_NODES = 10000
D = 128
NC = 2
NS = 16
NW = NC * NS
CHUNK = 128
N_PAD = 10240
ROWS_PER_TILE = N_PAD // NS
ZROWS = 64
GR = N_PAD // D
GPT = GR // NS
N_LINKS = 1024
LPW = N_LINKS // NW

_MESH = dict(core_axis_name="c", subcore_axis_name="s")


def _wid():
    return lax.axis_index("s") * NC + lax.axis_index("c")


def _agg_body(h_hbm, sd_hbm, zrow_hbm, out_hbm,
              isv, rows, zv, semg0, semg1, sems0, sems1, acc):
    k = sd_hbm.shape[1]
    c = lax.axis_index("c")
    s = lax.axis_index("s")
    w = _wid()
    base = s * ROWS_PER_TILE
    semg = (semg0, semg1)
    sems = (sems0, sems1)

    pltpu.sync_copy(zrow_hbm, zv)

    def zbody(r, carry):
        pltpu.sync_copy(zv, acc.at[pl.ds(base + r * ZROWS, ZROWS), :])
        return carry

    lax.fori_loop(0, ROWS_PER_TILE // ZROWS, zbody, 0)
    plsc.subcore_barrier()

    pltpu.sync_copy(sd_hbm.at[w, 0], isv.at[0])
    pltpu.async_copy(h_hbm.at[isv.at[0, 0]], rows.at[0], semg[0])

    def body(jh, carry):
        for b in range(2):
            nb = 1 - b
            j = jh * 2 + b
            pltpu.make_async_copy(
                h_hbm.at[isv.at[b, 0]], rows.at[b], semg[b]).wait()
            pltpu.async_copy(
                rows.at[b], acc.at[isv.at[b, 1]], sems[b], add=True)

            @pl.when(j > 0)
            def _():
                pltpu.make_async_copy(
                    rows.at[nb], acc.at[isv.at[nb, 1]], sems[nb]).wait()

            @pl.when(j + 1 < k)
            def _():
                pltpu.sync_copy(sd_hbm.at[w, j + 1], isv.at[nb])
                pltpu.async_copy(
                    h_hbm.at[isv.at[nb, 0]], rows.at[nb], semg[nb])
        return carry

    lax.fori_loop(0, k // 2, body, 0)
    pltpu.make_async_copy(
        rows.at[1], acc.at[isv.at[1, 1]], sems[1]).wait()
    plsc.subcore_barrier()

    def obody(r, carry):
        sl = pl.ds(base + r * ZROWS, ZROWS)
        pltpu.sync_copy(acc.at[sl, :], zv)
        pltpu.sync_copy(zv, out_hbm.at[c, sl, :])
        return carry

    lax.fori_loop(0, ROWS_PER_TILE // ZROWS, obody, 0)


def _link_body(h_hbm, sd_hbm, out_hbm, sdv, rows, prod, sem):
    w = _wid()
    pltpu.sync_copy(sd_hbm.at[w], sdv.at[0])
    pltpu.async_copy(h_hbm.at[sdv.at[0]], rows, sem).wait()
    for p in range(LPW):
        for t in range(D // 16):
            sl = pl.ds(t * 16, 16)
            prod[p, sl] = rows[p, sl] * rows[p + LPW, sl]
    pltpu.sync_copy(prod, out_hbm.at[pl.ds(w * LPW, LPW), :])


_RB = 1024


_EB = 4096


def _hist_body(dstr_ref, dstc_ref, o_ref, acc_ref):
    i = pl.program_id(0)

    @pl.when(i == 0)
    def _():
        acc_ref[...] = jnp.zeros_like(acc_ref)

    rr = dstr_ref[...]
    cc = dstc_ref[...]
    ri = lax.broadcasted_iota(jnp.int32, (1, GR), 1)
    ci = lax.broadcasted_iota(jnp.int32, (1, D), 1)
    R = (rr == ri).astype(jnp.bfloat16)
    C = (cc == ci).astype(jnp.bfloat16)
    acc_ref[...] += lax.dot_general(
        R, C, (((0,), (0,)), ((), ())),
        preferred_element_type=jnp.float32)

    @pl.when(i == pl.num_programs(0) - 1)
    def _():
        node = (lax.broadcasted_iota(jnp.int32, (GR, D), 0) * D
                + lax.broadcasted_iota(jnp.int32, (GR, D), 1))
        dg = acc_ref[...] + (node < N_NODES).astype(jnp.float32)
        o_ref[...] = jnp.where(dg > 0, lax.rsqrt(dg), 0.0)


def _mm1_body(x_ref, w_ref, dinv_ref, o_ref):
    i = pl.program_id(0)
    h = jnp.dot(x_ref[...], w_ref[...], preferred_element_type=jnp.float32)
    rows = i * _RB + lax.broadcasted_iota(jnp.int32, (_RB, 1), 0)
    o_ref[...] = jnp.where(rows < N_NODES, h * dinv_ref[...], 0.0)


def _mid_body(aggp_ref, dinv_ref, b1_ref, w2_ref, o_ref):
    i = pl.program_id(0)
    dinv = dinv_ref[...]
    agg = aggp_ref[0] + aggp_ref[1]
    h1 = jnp.maximum(agg * dinv + b1_ref[...], 0.0)
    h = jnp.dot(h1, w2_ref[...], preferred_element_type=jnp.float32) * dinv
    rows = i * _RB + lax.broadcasted_iota(jnp.int32, (_RB, 1), 0)
    o_ref[...] = jnp.where(rows < N_NODES, h, 0.0)


def _fin_body(aggp_ref, dinv_ref, b2_ref, o_ref):
    o_ref[...] = (aggp_ref[0] + aggp_ref[1]) * dinv_ref[...] + b2_ref[...]


def _head_body(link_ref, p1_ref, pb1_ref, p2_ref, pb2_ref, o_ref):
    t = jnp.dot(link_ref[...], p1_ref[...],
                preferred_element_type=jnp.float32) + pb1_ref[...]
    t = jnp.maximum(t, 0.0)
    o_ref[...] = jnp.dot(t, p2_ref[...],
                         preferred_element_type=jnp.float32) + pb2_ref[0, 0]


def kernel(x, edge_index, root_n_index, W1, b1, W2, b2, P1, pb1, P2, pb2):
    n = x.shape[0]
    e_real = edge_index.shape[1] + n
    k = -(-e_real // (NW * CHUNK))
    k += k % 2
    e_pad = NW * CHUNK * k
    n_extra = N_PAD - n

    loop = jnp.arange(n, dtype=jnp.int32)
    src = jnp.concatenate([edge_index[0].astype(jnp.int32), loop])
    dst = jnp.concatenate([edge_index[1].astype(jnp.int32), loop])
    pad_idx = n + (jnp.arange(e_pad - e_real, dtype=jnp.int32) % n_extra)
    src3 = jnp.concatenate([src, pad_idx]).reshape(NW, k, 1, CHUNK)
    dst3 = jnp.concatenate([dst, pad_idx]).reshape(NW, k, 1, CHUNK)
    sd3 = jnp.concatenate([src3, dst3], axis=2)

    x_pad = jnp.pad(x, ((0, n_extra), (0, 0)))
    zgrid = jnp.zeros((GR, D), jnp.float32)
    zrow = jnp.zeros((ZROWS, D), jnp.float32)
    iota80 = jnp.arange(GR, dtype=jnp.int32).reshape(1, GR)
    lpad = n + (jnp.arange(NW * 2 * LPW, dtype=jnp.int32) % n_extra)
    sd = jnp.concatenate(
        [root_n_index[:, 0].astype(jnp.int32).reshape(NW, LPW),
         root_n_index[:, 1].astype(jnp.int32).reshape(NW, LPW),
         lpad.reshape(NW, 2 * LPW)], axis=1)

    mesh = plsc.VectorSubcoreMesh(**_MESH)

    e_d = edge_index.shape[1]
    kd = -(-e_d // _EB)
    dpad = n + (jnp.arange(kd * _EB - e_d, dtype=jnp.int32) % n_extra)
    dd = jnp.concatenate([edge_index[1].astype(jnp.int32), dpad])
    dinv_grid = pl.pallas_call(
        _hist_body,
        grid=(kd,),
        in_specs=[
            pl.BlockSpec((_EB, 1), lambda i: (i, 0)),
            pl.BlockSpec((_EB, 1), lambda i: (i, 0)),
        ],
        out_specs=pl.BlockSpec((GR, D), lambda i: (0, 0)),
        out_shape=jax.ShapeDtypeStruct((GR, D), jnp.float32),
        scratch_shapes=[pltpu.VMEM((GR, D), jnp.float32)],
    )((dd >> 7).reshape(-1, 1), (dd & 127).reshape(-1, 1))
    dinv_col = dinv_grid.reshape(N_PAD, 1)

    agg_call = pl.kernel(
        _agg_body,
        out_type=jax.ShapeDtypeStruct((NC, N_PAD, D), jnp.float32),
        mesh=mesh,
        scratch_types=[
            pltpu.VMEM((2, 2, CHUNK), jnp.int32),
            pltpu.VMEM((2, CHUNK, D), jnp.float32),
            pltpu.VMEM((ZROWS, D), jnp.float32),
            pltpu.SemaphoreType.DMA,
            pltpu.SemaphoreType.DMA,
            pltpu.SemaphoreType.DMA,
            pltpu.SemaphoreType.DMA,
            pltpu.VMEM_SHARED((N_PAD, D), jnp.float32),
        ],
    )

    grid = (N_PAD // _RB,)
    h1p = pl.pallas_call(
        _mm1_body,
        grid=grid,
        in_specs=[
            pl.BlockSpec((_RB, D), lambda i: (i, 0)),
            pl.BlockSpec((D, D), lambda i: (0, 0)),
            pl.BlockSpec((_RB, 1), lambda i: (i, 0)),
        ],
        out_specs=pl.BlockSpec((_RB, D), lambda i: (i, 0)),
        out_shape=jax.ShapeDtypeStruct((N_PAD, D), jnp.float32),
    )(x_pad, W1, dinv_col)

    agg1 = agg_call(h1p, sd3, zrow)

    h2p = pl.pallas_call(
        _mid_body,
        grid=grid,
        in_specs=[
            pl.BlockSpec((NC, _RB, D), lambda i: (0, i, 0)),
            pl.BlockSpec((_RB, 1), lambda i: (i, 0)),
            pl.BlockSpec((1, D), lambda i: (0, 0)),
            pl.BlockSpec((D, D), lambda i: (0, 0)),
        ],
        out_specs=pl.BlockSpec((_RB, D), lambda i: (i, 0)),
        out_shape=jax.ShapeDtypeStruct((N_PAD, D), jnp.float32),
    )(agg1, dinv_col, b1.reshape(1, D), W2)

    agg2 = agg_call(h2p, sd3, zrow)

    h2 = pl.pallas_call(
        _fin_body,
        grid=grid,
        in_specs=[
            pl.BlockSpec((NC, _RB, D), lambda i: (0, i, 0)),
            pl.BlockSpec((_RB, 1), lambda i: (i, 0)),
            pl.BlockSpec((1, D), lambda i: (0, 0)),
        ],
        out_specs=pl.BlockSpec((_RB, D), lambda i: (i, 0)),
        out_shape=jax.ShapeDtypeStruct((N_PAD, D), jnp.float32),
    )(agg2, dinv_col, b2.reshape(1, D))

    link = pl.kernel(
        _link_body,
        out_type=jax.ShapeDtypeStruct((N_LINKS, D), jnp.float32),
        mesh=mesh,
        scratch_types=[
            pltpu.VMEM((1, 4 * LPW), jnp.int32),
            pltpu.VMEM((4 * LPW, D), jnp.float32),
            pltpu.VMEM((LPW, D), jnp.float32),
            pltpu.SemaphoreType.DMA,
        ],
    )(h2, sd)

    P2pad = jnp.pad(P2, ((0, 0), (0, D - P2.shape[1])))
    logits2d = pl.pallas_call(
        _head_body,
        in_specs=[
            pl.BlockSpec((N_LINKS, D), lambda: (0, 0)),
            pl.BlockSpec((D, D), lambda: (0, 0)),
            pl.BlockSpec((1, D), lambda: (0, 0)),
            pl.BlockSpec((D, D), lambda: (0, 0)),
            pl.BlockSpec((1, 1), lambda: (0, 0)),
        ],
        out_specs=pl.BlockSpec((N_LINKS, D), lambda: (0, 0)),
        out_shape=jax.ShapeDtypeStruct((N_LINKS, D), jnp.float32),
    )(link, P1, pb1.reshape(1, D), P2pad, pb2.reshape(1, 1))

    return logits2d[:, 0]


# speedup vs baseline: 13.2279x; 1.0950x over previous
import jax
import jax.numpy as jnp
from jax import lax
from jax.experimental import pallas as pl
from jax.experimental.pallas import tpu as pltpu
from jax.experimental.pallas import tpu_sc as plsc

N_NODES = 10000
D = 128
NC = 2
NS = 16
NW = NC * NS
CHUNK = 128
N_PAD = 10240
ROWS_PER_TILE = N_PAD // NS
ZROWS = 64
GR = N_PAD // D
N_LINKS = 1024
LPW = N_LINKS // NW

_MESH = dict(core_axis_name="c", subcore_axis_name="s")


def _wid():
    return lax.axis_index("s") * NC + lax.axis_index("c")


def _agg_body(h_hbm, sd_hbm, zrow_hbm, out_hbm,
              isv, rows, zv, semg0, semg1, sems0, sems1, semi0, semi1,
              semi2, semi3, acc):
    k = sd_hbm.shape[1]
    c = lax.axis_index("c")
    s = lax.axis_index("s")
    w = _wid()
    base = s * ROWS_PER_TILE
    semg = (semg0, semg1)
    sems = (sems0, sems1)
    semi = (semi0, semi1, semi2, semi3)

    pltpu.sync_copy(zrow_hbm, zv)

    def zbody(r, carry):
        pltpu.sync_copy(zv, acc.at[pl.ds(base + r * ZROWS, ZROWS), :])
        return carry

    lax.fori_loop(0, ROWS_PER_TILE // ZROWS, zbody, 0)
    plsc.subcore_barrier()

    pltpu.sync_copy(sd_hbm.at[w, 0], isv.at[0])
    pltpu.async_copy(h_hbm.at[isv.at[0, 0]], rows.at[0], semg[0])
    pltpu.async_copy(sd_hbm.at[w, 1], isv.at[1], semi[1])

    def body(jh, carry):
        for b4 in range(4):
            j = jh * 4 + b4
            b = b4 % 2
            nb = 1 - b
            b1 = (b4 + 1) % 4
            b2 = (b4 + 2) % 4
            bp = (b4 + 3) % 4
            pltpu.make_async_copy(
                h_hbm.at[isv.at[b4, 0]], rows.at[b], semg[b]).wait()
            pltpu.async_copy(
                rows.at[b], acc.at[isv.at[b4, 1]], sems[b], add=True)

            @pl.when(j > 0)
            def _():
                pltpu.make_async_copy(
                    rows.at[nb], acc.at[isv.at[bp, 1]], sems[nb]).wait()

            @pl.when(j + 1 < k)
            def _():
                pltpu.make_async_copy(
                    sd_hbm.at[w, j + 1], isv.at[b1], semi[b1]).wait()
                pltpu.async_copy(
                    h_hbm.at[isv.at[b1, 0]], rows.at[nb], semg[nb])

            @pl.when(j + 2 < k)
            def _():
                pltpu.async_copy(sd_hbm.at[w, j + 2], isv.at[b2], semi[b2])
        return carry

    lax.fori_loop(0, k // 4, body, 0)
    pltpu.make_async_copy(
        rows.at[1], acc.at[isv.at[(k - 1) % 4, 1]], sems[1]).wait()
    plsc.subcore_barrier()

    def obody(r, carry):
        sl = pl.ds(base + r * ZROWS, ZROWS)
        pltpu.sync_copy(acc.at[sl, :], zv)
        pltpu.sync_copy(zv, out_hbm.at[c, sl, :])
        return carry

    lax.fori_loop(0, ROWS_PER_TILE // ZROWS, obody, 0)


def _link_body(agg_hbm, sd_hbm, out_hbm, sdv, rows, acc_s, acc_d, sem):
    w = _wid()
    pltpu.sync_copy(sd_hbm.at[w], sdv.at[0])
    pltpu.async_copy(agg_hbm.at[sdv.at[0]], rows, sem).wait()
    for p in range(LPW):
        for t in range(D // 16):
            sl = pl.ds(t * 16, 16)
            acc_s[p, sl] = rows[p, sl] + rows[p + 2 * LPW, sl]
            acc_d[p, sl] = rows[p + LPW, sl] + rows[p + 3 * LPW, sl]
    pltpu.sync_copy(acc_s, out_hbm.at[0, pl.ds(w * LPW, LPW), :])
    pltpu.sync_copy(acc_d, out_hbm.at[1, pl.ds(w * LPW, LPW), :])


_RB = 1024
_EB = 4096


def _hist_body(dstr_ref, dstc_ref, o_ref, acc_ref):
    i = pl.program_id(0)

    @pl.when(i == 0)
    def _():
        acc_ref[...] = jnp.zeros_like(acc_ref)

    rr = dstr_ref[...]
    cc = dstc_ref[...]
    ri = lax.broadcasted_iota(jnp.int32, (1, GR), 1)
    ci = lax.broadcasted_iota(jnp.int32, (1, D), 1)
    R = (rr == ri).astype(jnp.bfloat16)
    C = (cc == ci).astype(jnp.bfloat16)
    acc_ref[...] += lax.dot_general(
        R, C, (((0,), (0,)), ((), ())),
        preferred_element_type=jnp.float32)

    @pl.when(i == pl.num_programs(0) - 1)
    def _():
        node = (lax.broadcasted_iota(jnp.int32, (GR, D), 0) * D
                + lax.broadcasted_iota(jnp.int32, (GR, D), 1))
        dg = acc_ref[...] + (node < N_NODES).astype(jnp.float32)
        o_ref[...] = jnp.where(dg > 0, lax.rsqrt(dg), 0.0)


def _mm1_body(x_ref, w_ref, dinv_ref, o_ref):
    i = pl.program_id(0)
    h = jnp.dot(x_ref[...], w_ref[...], preferred_element_type=jnp.float32)
    rows = i * _RB + lax.broadcasted_iota(jnp.int32, (_RB, 1), 0)
    o_ref[...] = jnp.where(rows < N_NODES, h * dinv_ref[...], 0.0)


def _mid_body(aggp_ref, dinv_ref, b1_ref, w2_ref, o_ref):
    i = pl.program_id(0)
    dinv = dinv_ref[...]
    agg = aggp_ref[0] + aggp_ref[1]
    h1 = jnp.maximum(agg * dinv + b1_ref[...], 0.0)
    h = jnp.dot(h1, w2_ref[...], preferred_element_type=jnp.float32) * dinv
    rows = i * _RB + lax.broadcasted_iota(jnp.int32, (_RB, 1), 0)
    o_ref[...] = jnp.where(rows < N_NODES, h, 0.0)


def _head_body(lag_ref, sr_ref, sc_ref, dr_ref, dc_ref, dg_ref, b2_ref,
               p1_ref, pb1_ref, p2_ref, pb2_ref, o_ref):
    dgrid = dg_ref[...]
    ri = lax.broadcasted_iota(jnp.int32, (1, GR), 1)
    ci = lax.broadcasted_iota(jnp.int32, (1, D), 1)

    def take_dinv(r_col, c_col):
        Rh = (r_col == ri).astype(jnp.float32)
        Ch = (c_col == ci).astype(jnp.float32)
        g = jnp.dot(Rh, dgrid, preferred_element_type=jnp.float32)
        return jnp.sum(g * Ch, axis=1, keepdims=True)

    dinv_s = take_dinv(sr_ref[...], sc_ref[...])
    dinv_d = take_dinv(dr_ref[...], dc_ref[...])
    h2s = lag_ref[0] * dinv_s + b2_ref[...]
    h2d = lag_ref[1] * dinv_d + b2_ref[...]
    t = jnp.dot(h2s * h2d, p1_ref[...],
                preferred_element_type=jnp.float32) + pb1_ref[...]
    t = jnp.maximum(t, 0.0)
    o_ref[...] = jnp.dot(t, p2_ref[...],
                         preferred_element_type=jnp.float32) + pb2_ref[0, 0]


def kernel(x, edge_index, root_n_index, W1, b1, W2, b2, P1, pb1, P2, pb2):
    n = x.shape[0]
    e_real = edge_index.shape[1] + n
    k = -(-e_real // (NW * CHUNK))
    k += (-k) % 4
    e_pad = NW * CHUNK * k
    n_extra = N_PAD - n

    loop = jnp.arange(n, dtype=jnp.int32)
    src = jnp.concatenate([edge_index[0].astype(jnp.int32), loop])
    dst = jnp.concatenate([edge_index[1].astype(jnp.int32), loop])
    pad_idx = n + (jnp.arange(e_pad - e_real, dtype=jnp.int32) % n_extra)
    src3 = jnp.concatenate([src, pad_idx]).reshape(NW, k, 1, CHUNK)
    dst3 = jnp.concatenate([dst, pad_idx]).reshape(NW, k, 1, CHUNK)
    sd3 = jnp.concatenate([src3, dst3], axis=2)

    x_pad = jnp.pad(x, ((0, n_extra), (0, 0)))
    zrow = jnp.zeros((ZROWS, D), jnp.float32)
    si = root_n_index[:, 0].astype(jnp.int32).reshape(NW, LPW)
    di = root_n_index[:, 1].astype(jnp.int32).reshape(NW, LPW)
    sd = jnp.concatenate([si, di, si + N_PAD, di + N_PAD], axis=1)

    mesh = plsc.VectorSubcoreMesh(**_MESH)

    e_d = edge_index.shape[1]
    kd = -(-e_d // _EB)
    dpad = n + (jnp.arange(kd * _EB - e_d, dtype=jnp.int32) % n_extra)
    dd = jnp.concatenate([edge_index[1].astype(jnp.int32), dpad])
    dinv_grid = pl.pallas_call(
        _hist_body,
        grid=(kd,),
        in_specs=[
            pl.BlockSpec((_EB, 1), lambda i: (i, 0)),
            pl.BlockSpec((_EB, 1), lambda i: (i, 0)),
        ],
        out_specs=pl.BlockSpec((GR, D), lambda i: (0, 0)),
        out_shape=jax.ShapeDtypeStruct((GR, D), jnp.float32),
        scratch_shapes=[pltpu.VMEM((GR, D), jnp.float32)],
    )((dd >> 7).reshape(-1, 1), (dd & 127).reshape(-1, 1))
    dinv_col = dinv_grid.reshape(N_PAD, 1)

    agg_call = pl.kernel(
        _agg_body,
        out_type=jax.ShapeDtypeStruct((NC, N_PAD, D), jnp.float32),
        mesh=mesh,
        scratch_types=[
            pltpu.VMEM((4, 2, CHUNK), jnp.int32),
            pltpu.VMEM((2, CHUNK, D), jnp.float32),
            pltpu.VMEM((ZROWS, D), jnp.float32),
            pltpu.SemaphoreType.DMA,
            pltpu.SemaphoreType.DMA,
            pltpu.SemaphoreType.DMA,
            pltpu.SemaphoreType.DMA,
            pltpu.SemaphoreType.DMA,
            pltpu.SemaphoreType.DMA,
            pltpu.SemaphoreType.DMA,
            pltpu.SemaphoreType.DMA,
            pltpu.VMEM_SHARED((N_PAD, D), jnp.float32),
        ],
    )

    grid = (N_PAD // _RB,)
    h1p = pl.pallas_call(
        _mm1_body,
        grid=grid,
        in_specs=[
            pl.BlockSpec((_RB, D), lambda i: (i, 0)),
            pl.BlockSpec((D, D), lambda i: (0, 0)),
            pl.BlockSpec((_RB, 1), lambda i: (i, 0)),
        ],
        out_specs=pl.BlockSpec((_RB, D), lambda i: (i, 0)),
        out_shape=jax.ShapeDtypeStruct((N_PAD, D), jnp.float32),
    )(x_pad, W1, dinv_col)

    agg1 = agg_call(h1p, sd3, zrow)

    h2p = pl.pallas_call(
        _mid_body,
        grid=grid,
        in_specs=[
            pl.BlockSpec((NC, _RB, D), lambda i: (0, i, 0)),
            pl.BlockSpec((_RB, 1), lambda i: (i, 0)),
            pl.BlockSpec((1, D), lambda i: (0, 0)),
            pl.BlockSpec((D, D), lambda i: (0, 0)),
        ],
        out_specs=pl.BlockSpec((_RB, D), lambda i: (i, 0)),
        out_shape=jax.ShapeDtypeStruct((N_PAD, D), jnp.float32),
    )(agg1, dinv_col, b1.reshape(1, D), W2)

    agg2 = agg_call(h2p, sd3, zrow)

    lagg = pl.kernel(
        _link_body,
        out_type=jax.ShapeDtypeStruct((2, N_LINKS, D), jnp.float32),
        mesh=mesh,
        scratch_types=[
            pltpu.VMEM((1, 4 * LPW), jnp.int32),
            pltpu.VMEM((4 * LPW, D), jnp.float32),
            pltpu.VMEM((LPW, D), jnp.float32),
            pltpu.VMEM((LPW, D), jnp.float32),
            pltpu.SemaphoreType.DMA,
        ],
    )(agg2.reshape(NC * N_PAD, D), sd)

    P2pad = jnp.pad(P2, ((0, 0), (0, D - P2.shape[1])))
    sl_ = root_n_index[:, 0].astype(jnp.int32).reshape(N_LINKS, 1)
    dl_ = root_n_index[:, 1].astype(jnp.int32).reshape(N_LINKS, 1)
    logits2d = pl.pallas_call(
        _head_body,
        in_specs=[
            pl.BlockSpec((2, N_LINKS, D), lambda: (0, 0, 0)),
            pl.BlockSpec((N_LINKS, 1), lambda: (0, 0)),
            pl.BlockSpec((N_LINKS, 1), lambda: (0, 0)),
            pl.BlockSpec((N_LINKS, 1), lambda: (0, 0)),
            pl.BlockSpec((N_LINKS, 1), lambda: (0, 0)),
            pl.BlockSpec((GR, D), lambda: (0, 0)),
            pl.BlockSpec((1, D), lambda: (0, 0)),
            pl.BlockSpec((D, D), lambda: (0, 0)),
            pl.BlockSpec((1, D), lambda: (0, 0)),
            pl.BlockSpec((D, D), lambda: (0, 0)),
            pl.BlockSpec((1, 1), lambda: (0, 0)),
        ],
        out_specs=pl.BlockSpec((N_LINKS, D), lambda: (0, 0)),
        out_shape=jax.ShapeDtypeStruct((N_LINKS, D), jnp.float32),
    )(lagg, sl_ >> 7, sl_ & 127, dl_ >> 7, dl_ & 127, dinv_grid,
      b2.reshape(1, D), P1, pb1.reshape(1, D), P2pad, pb2.reshape(1, 1))

    return logits2d[:, 0]


# speedup vs baseline: 19.5582x; 1.4786x over previous
import jax
import jax.numpy as jnp
from jax import lax
from jax.experimental import pallas as pl
from jax.experimental.pallas import tpu as pltpu
from jax.experimental.pallas import tpu_sc as plsc

N_NODES = 10000
D = 128
NC = 2
NS = 16
NW = NC * NS
CHUNK = 128
N_PAD = 10240
ROWS_PER_TILE = N_PAD // NS
ZROWS = 64
GR = N_PAD // D
N_LINKS = 1024
LPW = N_LINKS // NW

_MESH = dict(core_axis_name="c", subcore_axis_name="s")


def _wid():
    return lax.axis_index("s") * NC + lax.axis_index("c")


def _agg_body(h_hbm, sd_hbm, zrow_hbm, out_hbm,
              isv, rows, zv, semg0, semg1, sems0, sems1, semi0, semi1,
              semi2, semi3, acc):
    k = sd_hbm.shape[1]
    c = lax.axis_index("c")
    s = lax.axis_index("s")
    w = _wid()
    base = s * ROWS_PER_TILE
    semg = (semg0, semg1)
    sems = (sems0, sems1)
    semi = (semi0, semi1, semi2, semi3)

    pltpu.sync_copy(zrow_hbm, zv)

    def zbody(r, carry):
        pltpu.sync_copy(zv, acc.at[pl.ds(base + r * ZROWS, ZROWS), :])
        return carry

    lax.fori_loop(0, ROWS_PER_TILE // ZROWS, zbody, 0)
    plsc.subcore_barrier()

    pltpu.sync_copy(sd_hbm.at[w, 0], isv.at[0])
    pltpu.async_copy(h_hbm.at[isv.at[0, 0]], rows.at[0], semg[0])
    pltpu.async_copy(sd_hbm.at[w, 1], isv.at[1], semi[1])

    def body(jh, carry):
        for b4 in range(4):
            j = jh * 4 + b4
            b = b4 % 2
            nb = 1 - b
            b1 = (b4 + 1) % 4
            b2 = (b4 + 2) % 4
            bp = (b4 + 3) % 4
            pltpu.make_async_copy(
                h_hbm.at[isv.at[b4, 0]], rows.at[b], semg[b]).wait()
            pltpu.async_copy(
                rows.at[b], acc.at[isv.at[b4, 1]], sems[b], add=True)

            @pl.when(j > 0)
            def _():
                pltpu.make_async_copy(
                    rows.at[nb], acc.at[isv.at[bp, 1]], sems[nb]).wait()

            @pl.when(j + 1 < k)
            def _():
                pltpu.make_async_copy(
                    sd_hbm.at[w, j + 1], isv.at[b1], semi[b1]).wait()
                pltpu.async_copy(
                    h_hbm.at[isv.at[b1, 0]], rows.at[nb], semg[nb])

            @pl.when(j + 2 < k)
            def _():
                pltpu.async_copy(sd_hbm.at[w, j + 2], isv.at[b2], semi[b2])
        return carry

    lax.fori_loop(0, k // 4, body, 0)
    pltpu.make_async_copy(
        rows.at[1], acc.at[isv.at[(k - 1) % 4, 1]], sems[1]).wait()
    plsc.subcore_barrier()

    def obody(r, carry):
        sl = pl.ds(base + r * ZROWS, ZROWS)
        pltpu.sync_copy(acc.at[sl, :], zv)
        pltpu.sync_copy(zv, out_hbm.at[c, sl, :])
        return carry

    lax.fori_loop(0, ROWS_PER_TILE // ZROWS, obody, 0)


def _link_body(agg_hbm, deg_hbm, sd_hbm, out_hbm, sdv, rows, rowsd,
               acc_s, acc_d, deg_s, deg_d, sem):
    w = _wid()
    pltpu.sync_copy(sd_hbm.at[w], sdv.at[0])
    pltpu.async_copy(agg_hbm.at[sdv.at[0]], rows, sem).wait()
    pltpu.async_copy(deg_hbm.at[sdv.at[0]], rowsd, sem).wait()
    for p in range(LPW):
        for t in range(D // 16):
            sl = pl.ds(t * 16, 16)
            acc_s[p, sl] = rows[p, sl] + rows[p + 2 * LPW, sl]
            acc_d[p, sl] = rows[p + LPW, sl] + rows[p + 3 * LPW, sl]
            deg_s[p, sl] = rowsd[p, sl] + rowsd[p + 2 * LPW, sl]
            deg_d[p, sl] = rowsd[p + LPW, sl] + rowsd[p + 3 * LPW, sl]
    pltpu.sync_copy(acc_s, out_hbm.at[0, pl.ds(w * LPW, LPW), :])
    pltpu.sync_copy(acc_d, out_hbm.at[1, pl.ds(w * LPW, LPW), :])
    pltpu.sync_copy(deg_s, out_hbm.at[2, pl.ds(w * LPW, LPW), :])
    pltpu.sync_copy(deg_d, out_hbm.at[3, pl.ds(w * LPW, LPW), :])


_RB = 1024

def _prep_body(aggd_ref, o_ref):
    dg = aggd_ref[0, :, 0:1] + aggd_ref[1, :, 0:1]
    o_ref[...] = jnp.where(dg > 0, lax.rsqrt(dg), 0.0)


def _mm1_body(x_ref, w_ref, dinv_ref, o_ref):
    i = pl.program_id(0)
    h = jnp.dot(x_ref[...], w_ref[...], preferred_element_type=jnp.float32)
    rows = i * _RB + lax.broadcasted_iota(jnp.int32, (_RB, 1), 0)
    o_ref[...] = jnp.where(rows < N_NODES, h * dinv_ref[...], 0.0)


def _mid_body(aggp_ref, dinv_ref, b1_ref, w2_ref, o_ref):
    i = pl.program_id(0)
    dinv = dinv_ref[...]
    agg = aggp_ref[0] + aggp_ref[1]
    h1 = jnp.maximum(agg * dinv + b1_ref[...], 0.0)
    h = jnp.dot(h1, w2_ref[...], preferred_element_type=jnp.float32) * dinv
    rows = i * _RB + lax.broadcasted_iota(jnp.int32, (_RB, 1), 0)
    o_ref[...] = jnp.where(rows < N_NODES, h, 0.0)


def _head_body(lag_ref, b2_ref, p1_ref, pb1_ref, p2_ref, pb2_ref, o_ref):
    def dinv_of(degrow):
        return jnp.where(degrow > 0, lax.rsqrt(degrow), 0.0)

    h2s = lag_ref[0] * dinv_of(lag_ref[2, :, 0:1]) + b2_ref[...]
    h2d = lag_ref[1] * dinv_of(lag_ref[3, :, 0:1]) + b2_ref[...]
    t = jnp.dot(h2s * h2d, p1_ref[...],
                preferred_element_type=jnp.float32) + pb1_ref[...]
    t = jnp.maximum(t, 0.0)
    o_ref[...] = jnp.dot(t, p2_ref[...],
                         preferred_element_type=jnp.float32) + pb2_ref[0, 0]


def kernel(x, edge_index, root_n_index, W1, b1, W2, b2, P1, pb1, P2, pb2):
    n = x.shape[0]
    e_real = edge_index.shape[1] + n
    k = -(-e_real // (NW * CHUNK))
    k += (-k) % 4
    e_pad = NW * CHUNK * k
    n_extra = N_PAD - n

    loop = jnp.arange(n, dtype=jnp.int32)
    src = jnp.concatenate([edge_index[0].astype(jnp.int32), loop])
    dst = jnp.concatenate([edge_index[1].astype(jnp.int32), loop])
    pad_idx = n + (jnp.arange(e_pad - e_real, dtype=jnp.int32) % n_extra)
    src3 = jnp.concatenate([src, pad_idx]).reshape(NW, k, 1, CHUNK)
    dst3 = jnp.concatenate([dst, pad_idx]).reshape(NW, k, 1, CHUNK)
    sd3 = jnp.concatenate([src3, dst3], axis=2)

    x_pad = jnp.pad(x, ((0, n_extra), (0, 0)))
    zrow = jnp.zeros((ZROWS, D), jnp.float32)
    si = root_n_index[:, 0].astype(jnp.int32).reshape(NW, LPW)
    di = root_n_index[:, 1].astype(jnp.int32).reshape(NW, LPW)
    sd = jnp.concatenate([si, di, si + N_PAD, di + N_PAD], axis=1)

    mesh = plsc.VectorSubcoreMesh(**_MESH)


    agg_call = pl.kernel(
        _agg_body,
        out_type=jax.ShapeDtypeStruct((NC, N_PAD, D), jnp.float32),
        mesh=mesh,
        scratch_types=[
            pltpu.VMEM((4, 2, CHUNK), jnp.int32),
            pltpu.VMEM((2, CHUNK, D), jnp.float32),
            pltpu.VMEM((ZROWS, D), jnp.float32),
            pltpu.SemaphoreType.DMA,
            pltpu.SemaphoreType.DMA,
            pltpu.SemaphoreType.DMA,
            pltpu.SemaphoreType.DMA,
            pltpu.SemaphoreType.DMA,
            pltpu.SemaphoreType.DMA,
            pltpu.SemaphoreType.DMA,
            pltpu.SemaphoreType.DMA,
            pltpu.VMEM_SHARED((N_PAD, D), jnp.float32),
        ],
    )

    ones_pad = jnp.ones((N_PAD, D), jnp.float32)
    sdd3 = jnp.concatenate([dst3, dst3], axis=2)
    aggd = agg_call(ones_pad, sdd3, zrow)

    dinv_col = pl.pallas_call(
        _prep_body,
        grid=(N_PAD // _RB,),
        in_specs=[pl.BlockSpec((NC, _RB, D), lambda i: (0, i, 0))],
        out_specs=pl.BlockSpec((_RB, 1), lambda i: (i, 0)),
        out_shape=jax.ShapeDtypeStruct((N_PAD, 1), jnp.float32),
    )(aggd)

    grid = (N_PAD // _RB,)
    h1p = pl.pallas_call(
        _mm1_body,
        grid=grid,
        in_specs=[
            pl.BlockSpec((_RB, D), lambda i: (i, 0)),
            pl.BlockSpec((D, D), lambda i: (0, 0)),
            pl.BlockSpec((_RB, 1), lambda i: (i, 0)),
        ],
        out_specs=pl.BlockSpec((_RB, D), lambda i: (i, 0)),
        out_shape=jax.ShapeDtypeStruct((N_PAD, D), jnp.float32),
    )(x_pad, W1, dinv_col)

    agg1 = agg_call(h1p, sd3, zrow)

    h2p = pl.pallas_call(
        _mid_body,
        grid=grid,
        in_specs=[
            pl.BlockSpec((NC, _RB, D), lambda i: (0, i, 0)),
            pl.BlockSpec((_RB, 1), lambda i: (i, 0)),
            pl.BlockSpec((1, D), lambda i: (0, 0)),
            pl.BlockSpec((D, D), lambda i: (0, 0)),
        ],
        out_specs=pl.BlockSpec((_RB, D), lambda i: (i, 0)),
        out_shape=jax.ShapeDtypeStruct((N_PAD, D), jnp.float32),
    )(agg1, dinv_col, b1.reshape(1, D), W2)

    agg2 = agg_call(h2p, sd3, zrow)

    lagg = pl.kernel(
        _link_body,
        out_type=jax.ShapeDtypeStruct((4, N_LINKS, D), jnp.float32),
        mesh=mesh,
        scratch_types=[
            pltpu.VMEM((1, 4 * LPW), jnp.int32),
            pltpu.VMEM((4 * LPW, D), jnp.float32),
            pltpu.VMEM((4 * LPW, D), jnp.float32),
            pltpu.VMEM((LPW, D), jnp.float32),
            pltpu.VMEM((LPW, D), jnp.float32),
            pltpu.VMEM((LPW, D), jnp.float32),
            pltpu.VMEM((LPW, D), jnp.float32),
            pltpu.SemaphoreType.DMA,
        ],
    )(agg2.reshape(NC * N_PAD, D), aggd.reshape(NC * N_PAD, D), sd)

    P2pad = jnp.pad(P2, ((0, 0), (0, D - P2.shape[1])))
    logits2d = pl.pallas_call(
        _head_body,
        in_specs=[
            pl.BlockSpec((4, N_LINKS, D), lambda: (0, 0, 0)),
            pl.BlockSpec((1, D), lambda: (0, 0)),
            pl.BlockSpec((D, D), lambda: (0, 0)),
            pl.BlockSpec((1, D), lambda: (0, 0)),
            pl.BlockSpec((D, D), lambda: (0, 0)),
            pl.BlockSpec((1, 1), lambda: (0, 0)),
        ],
        out_specs=pl.BlockSpec((N_LINKS, D), lambda: (0, 0)),
        out_shape=jax.ShapeDtypeStruct((N_LINKS, D), jnp.float32),
    )(lagg, b2.reshape(1, D), P1, pb1.reshape(1, D), P2pad, pb2.reshape(1, 1))

    return logits2d[:, 0]


# speedup vs baseline: 22.5800x; 1.1545x over previous
import jax
import jax.numpy as jnp
from jax import lax
from jax.experimental import pallas as pl
from jax.experimental.pallas import tpu as pltpu
from jax.experimental.pallas import tpu_sc as plsc

N_NODES = 10000
D = 128
NC = 2
NS = 16
NW = NC * NS
CHUNK = 128
N_PAD = 10240
ROWS_PER_TILE = N_PAD // NS
ZROWS = 64
GR = N_PAD // D
N_LINKS = 1024
LPW = N_LINKS // NW

_MESH = dict(core_axis_name="c", subcore_axis_name="s")


def _wid():
    return lax.axis_index("s") * NC + lax.axis_index("c")


def _agg_body(h_hbm, sd_hbm, zrow_hbm, out_hbm,
              isv, rows, zv, semg0, semg1, sems0, sems1, semi0, semi1,
              semi2, semi3, acc):
    k = sd_hbm.shape[1]
    c = lax.axis_index("c")
    s = lax.axis_index("s")
    w = _wid()
    base = s * ROWS_PER_TILE
    semg = (semg0, semg1)
    sems = (sems0, sems1)
    semi = (semi0, semi1, semi2, semi3)

    pltpu.sync_copy(zrow_hbm, zv)

    def zbody(r, carry):
        pltpu.sync_copy(zv, acc.at[pl.ds(base + r * ZROWS, ZROWS), :])
        return carry

    lax.fori_loop(0, ROWS_PER_TILE // ZROWS, zbody, 0)
    plsc.subcore_barrier()

    pltpu.sync_copy(sd_hbm.at[w, 0], isv.at[0])
    pltpu.async_copy(h_hbm.at[isv.at[0, 0]], rows.at[0], semg[0])
    pltpu.async_copy(sd_hbm.at[w, 1], isv.at[1], semi[1])

    def body(jh, carry):
        for b4 in range(4):
            j = jh * 4 + b4
            b = b4 % 2
            nb = 1 - b
            b1 = (b4 + 1) % 4
            b2 = (b4 + 2) % 4
            bp = (b4 + 3) % 4
            pltpu.make_async_copy(
                h_hbm.at[isv.at[b4, 0]], rows.at[b], semg[b]).wait()
            pltpu.async_copy(
                rows.at[b], acc.at[isv.at[b4, 1]], sems[b], add=True)

            @pl.when(j > 0)
            def _():
                pltpu.make_async_copy(
                    rows.at[nb], acc.at[isv.at[bp, 1]], sems[nb]).wait()

            @pl.when(j + 1 < k)
            def _():
                pltpu.make_async_copy(
                    sd_hbm.at[w, j + 1], isv.at[b1], semi[b1]).wait()
                pltpu.async_copy(
                    h_hbm.at[isv.at[b1, 0]], rows.at[nb], semg[nb])

            @pl.when(j + 2 < k)
            def _():
                pltpu.async_copy(sd_hbm.at[w, j + 2], isv.at[b2], semi[b2])
        return carry

    lax.fori_loop(0, k // 4, body, 0)
    pltpu.make_async_copy(
        rows.at[1], acc.at[isv.at[(k - 1) % 4, 1]], sems[1]).wait()
    plsc.subcore_barrier()

    def obody(r, carry):
        sl = pl.ds(base + r * ZROWS, ZROWS)
        pltpu.sync_copy(acc.at[sl, :], zv)
        pltpu.sync_copy(zv, out_hbm.at[c, sl, :])
        return carry

    lax.fori_loop(0, ROWS_PER_TILE // ZROWS, obody, 0)


def _deg_body(ones_hbm, d_hbm, zrow_hbm, out_hbm,
              idxv, ones_v, zv, semd0, semd1, semi0, semi1, acc):
    k = d_hbm.shape[1]
    c = lax.axis_index("c")
    s = lax.axis_index("s")
    w = _wid()
    base = s * ROWS_PER_TILE
    semd = (semd0, semd1)
    semi = (semi0, semi1)

    pltpu.sync_copy(zrow_hbm, zv)

    def zbody(r, carry):
        pltpu.sync_copy(zv, acc.at[pl.ds(base + r * ZROWS, ZROWS), :])
        return carry

    lax.fori_loop(0, ROWS_PER_TILE // ZROWS, zbody, 0)
    pltpu.sync_copy(ones_hbm, ones_v)
    plsc.subcore_barrier()

    pltpu.sync_copy(d_hbm.at[w, 0], idxv.at[0])

    def body(jh, carry):
        for b in range(2):
            nb = 1 - b
            j = jh * 2 + b

            @pl.when(j > 0)
            def _():
                pltpu.make_async_copy(
                    d_hbm.at[w, j], idxv.at[b], semi[b]).wait()

            pltpu.async_copy(
                ones_v, acc.at[idxv.at[b, 0]], semd[b], add=True)

            @pl.when(j > 0)
            def _():
                pltpu.make_async_copy(
                    ones_v, acc.at[idxv.at[nb, 0]], semd[nb]).wait()

            @pl.when(j + 1 < k)
            def _():
                pltpu.async_copy(d_hbm.at[w, j + 1], idxv.at[nb], semi[nb])
        return carry

    lax.fori_loop(0, k // 2, body, 0)
    pltpu.make_async_copy(ones_v, acc.at[idxv.at[1, 0]], semd[1]).wait()
    plsc.subcore_barrier()

    def obody(r, carry):
        sl = pl.ds(base + r * ZROWS, ZROWS)
        pltpu.sync_copy(acc.at[sl, :], zv)
        pltpu.sync_copy(zv, out_hbm.at[c, sl, :])
        return carry

    lax.fori_loop(0, ROWS_PER_TILE // ZROWS, obody, 0)


def _link_body(agg_hbm, deg_hbm, sd_hbm, out_hbm, sdv, rows, rowsd,
               acc_s, acc_d, deg_s, deg_d, sem):
    w = _wid()
    pltpu.sync_copy(sd_hbm.at[w], sdv.at[0])
    pltpu.async_copy(agg_hbm.at[sdv.at[0]], rows, sem).wait()
    pltpu.async_copy(deg_hbm.at[sdv.at[0]], rowsd, sem).wait()
    for p in range(LPW):
        for t in range(D // 16):
            sl = pl.ds(t * 16, 16)
            acc_s[p, sl] = rows[p, sl] + rows[p + 2 * LPW, sl]
            acc_d[p, sl] = rows[p + LPW, sl] + rows[p + 3 * LPW, sl]
            deg_s[p, sl] = rowsd[p, sl] + rowsd[p + 2 * LPW, sl]
            deg_d[p, sl] = rowsd[p + LPW, sl] + rowsd[p + 3 * LPW, sl]
    pltpu.sync_copy(acc_s, out_hbm.at[0, pl.ds(w * LPW, LPW), :])
    pltpu.sync_copy(acc_d, out_hbm.at[1, pl.ds(w * LPW, LPW), :])
    pltpu.sync_copy(deg_s, out_hbm.at[2, pl.ds(w * LPW, LPW), :])
    pltpu.sync_copy(deg_d, out_hbm.at[3, pl.ds(w * LPW, LPW), :])


_RB = 1024

def _prep_body(aggd_ref, o_ref):
    dg = aggd_ref[0, :, 0:1] + aggd_ref[1, :, 0:1]
    o_ref[...] = jnp.where(dg > 0, lax.rsqrt(dg), 0.0)


def _mm1_body(x_ref, w_ref, dinv_ref, o_ref):
    i = pl.program_id(0)
    h = jnp.dot(x_ref[...], w_ref[...], preferred_element_type=jnp.float32)
    rows = i * _RB + lax.broadcasted_iota(jnp.int32, (_RB, 1), 0)
    o_ref[...] = jnp.where(rows < N_NODES, h * dinv_ref[...], 0.0)


def _mid_body(aggp_ref, dinv_ref, b1_ref, w2_ref, o_ref):
    i = pl.program_id(0)
    dinv = dinv_ref[...]
    agg = aggp_ref[0] + aggp_ref[1]
    h1 = jnp.maximum(agg * dinv + b1_ref[...], 0.0)
    h = jnp.dot(h1, w2_ref[...], preferred_element_type=jnp.float32) * dinv
    rows = i * _RB + lax.broadcasted_iota(jnp.int32, (_RB, 1), 0)
    o_ref[...] = jnp.where(rows < N_NODES, h, 0.0)


def _head_body(lag_ref, b2_ref, p1_ref, pb1_ref, p2_ref, pb2_ref, o_ref):
    def dinv_of(degrow):
        return jnp.where(degrow > 0, lax.rsqrt(degrow), 0.0)

    h2s = lag_ref[0] * dinv_of(lag_ref[2, :, 0:1]) + b2_ref[...]
    h2d = lag_ref[1] * dinv_of(lag_ref[3, :, 0:1]) + b2_ref[...]
    t = jnp.dot(h2s * h2d, p1_ref[...],
                preferred_element_type=jnp.float32) + pb1_ref[...]
    t = jnp.maximum(t, 0.0)
    o_ref[...] = jnp.dot(t, p2_ref[...],
                         preferred_element_type=jnp.float32) + pb2_ref[0, 0]


def kernel(x, edge_index, root_n_index, W1, b1, W2, b2, P1, pb1, P2, pb2):
    n = x.shape[0]
    e_real = edge_index.shape[1] + n
    k = -(-e_real // (NW * CHUNK))
    k += (-k) % 4
    e_pad = NW * CHUNK * k
    n_extra = N_PAD - n

    loop = jnp.arange(n, dtype=jnp.int32)
    src = jnp.concatenate([edge_index[0].astype(jnp.int32), loop])
    dst = jnp.concatenate([edge_index[1].astype(jnp.int32), loop])
    pad_idx = n + (jnp.arange(e_pad - e_real, dtype=jnp.int32) % n_extra)
    src3 = jnp.concatenate([src, pad_idx]).reshape(NW, k, 1, CHUNK)
    dst3 = jnp.concatenate([dst, pad_idx]).reshape(NW, k, 1, CHUNK)
    sd3 = jnp.concatenate([src3, dst3], axis=2)

    x_pad = jnp.pad(x, ((0, n_extra), (0, 0)))
    zrow = jnp.zeros((ZROWS, D), jnp.float32)
    si = root_n_index[:, 0].astype(jnp.int32).reshape(NW, LPW)
    di = root_n_index[:, 1].astype(jnp.int32).reshape(NW, LPW)
    sd = jnp.concatenate([si, di, si + N_PAD, di + N_PAD], axis=1)

    mesh = plsc.VectorSubcoreMesh(**_MESH)


    agg_call = pl.kernel(
        _agg_body,
        out_type=jax.ShapeDtypeStruct((NC, N_PAD, D), jnp.float32),
        mesh=mesh,
        scratch_types=[
            pltpu.VMEM((4, 2, CHUNK), jnp.int32),
            pltpu.VMEM((2, CHUNK, D), jnp.float32),
            pltpu.VMEM((ZROWS, D), jnp.float32),
            pltpu.SemaphoreType.DMA,
            pltpu.SemaphoreType.DMA,
            pltpu.SemaphoreType.DMA,
            pltpu.SemaphoreType.DMA,
            pltpu.SemaphoreType.DMA,
            pltpu.SemaphoreType.DMA,
            pltpu.SemaphoreType.DMA,
            pltpu.SemaphoreType.DMA,
            pltpu.VMEM_SHARED((N_PAD, D), jnp.float32),
        ],
    )

    ones_rows = jnp.ones((CHUNK, D), jnp.float32)
    aggd = pl.kernel(
        _deg_body,
        out_type=jax.ShapeDtypeStruct((NC, N_PAD, D), jnp.float32),
        mesh=mesh,
        scratch_types=[
            pltpu.VMEM((2, 1, CHUNK), jnp.int32),
            pltpu.VMEM((CHUNK, D), jnp.float32),
            pltpu.VMEM((ZROWS, D), jnp.float32),
            pltpu.SemaphoreType.DMA,
            pltpu.SemaphoreType.DMA,
            pltpu.SemaphoreType.DMA,
            pltpu.SemaphoreType.DMA,
            pltpu.VMEM_SHARED((N_PAD, D), jnp.float32),
        ],
    )(ones_rows, dst3, zrow)

    dinv_col = pl.pallas_call(
        _prep_body,
        grid=(N_PAD // _RB,),
        in_specs=[pl.BlockSpec((NC, _RB, D), lambda i: (0, i, 0))],
        out_specs=pl.BlockSpec((_RB, 1), lambda i: (i, 0)),
        out_shape=jax.ShapeDtypeStruct((N_PAD, 1), jnp.float32),
    )(aggd)

    grid = (N_PAD // _RB,)
    h1p = pl.pallas_call(
        _mm1_body,
        grid=grid,
        in_specs=[
            pl.BlockSpec((_RB, D), lambda i: (i, 0)),
            pl.BlockSpec((D, D), lambda i: (0, 0)),
            pl.BlockSpec((_RB, 1), lambda i: (i, 0)),
        ],
        out_specs=pl.BlockSpec((_RB, D), lambda i: (i, 0)),
        out_shape=jax.ShapeDtypeStruct((N_PAD, D), jnp.float32),
    )(x_pad, W1, dinv_col)

    agg1 = agg_call(h1p, sd3, zrow)

    h2p = pl.pallas_call(
        _mid_body,
        grid=grid,
        in_specs=[
            pl.BlockSpec((NC, _RB, D), lambda i: (0, i, 0)),
            pl.BlockSpec((_RB, 1), lambda i: (i, 0)),
            pl.BlockSpec((1, D), lambda i: (0, 0)),
            pl.BlockSpec((D, D), lambda i: (0, 0)),
        ],
        out_specs=pl.BlockSpec((_RB, D), lambda i: (i, 0)),
        out_shape=jax.ShapeDtypeStruct((N_PAD, D), jnp.float32),
    )(agg1, dinv_col, b1.reshape(1, D), W2)

    agg2 = agg_call(h2p, sd3, zrow)

    lagg = pl.kernel(
        _link_body,
        out_type=jax.ShapeDtypeStruct((4, N_LINKS, D), jnp.float32),
        mesh=mesh,
        scratch_types=[
            pltpu.VMEM((1, 4 * LPW), jnp.int32),
            pltpu.VMEM((4 * LPW, D), jnp.float32),
            pltpu.VMEM((4 * LPW, D), jnp.float32),
            pltpu.VMEM((LPW, D), jnp.float32),
            pltpu.VMEM((LPW, D), jnp.float32),
            pltpu.VMEM((LPW, D), jnp.float32),
            pltpu.VMEM((LPW, D), jnp.float32),
            pltpu.SemaphoreType.DMA,
        ],
    )(agg2.reshape(NC * N_PAD, D), aggd.reshape(NC * N_PAD, D), sd)

    P2pad = jnp.pad(P2, ((0, 0), (0, D - P2.shape[1])))
    logits2d = pl.pallas_call(
        _head_body,
        in_specs=[
            pl.BlockSpec((4, N_LINKS, D), lambda: (0, 0, 0)),
            pl.BlockSpec((1, D), lambda: (0, 0)),
            pl.BlockSpec((D, D), lambda: (0, 0)),
            pl.BlockSpec((1, D), lambda: (0, 0)),
            pl.BlockSpec((D, D), lambda: (0, 0)),
            pl.BlockSpec((1, 1), lambda: (0, 0)),
        ],
        out_specs=pl.BlockSpec((N_LINKS, D), lambda: (0, 0)),
        out_shape=jax.ShapeDtypeStruct((N_LINKS, D), jnp.float32),
    )(lagg, b2.reshape(1, D), P1, pb1.reshape(1, D), P2pad, pb2.reshape(1, 1))

    return logits2d[:, 0]
